# baseline (speedup 1.0000x reference)
import functools

import jax
import jax.numpy as jnp
from jax import lax
from jax.experimental import pallas as pl
from jax.experimental.pallas import tpu as pltpu
from jax.experimental.pallas import tpu_sc as plsc

NC = 2
NS = 16
L = 16
NW = NC * NS
B = 16384
D = 64
BPW = B // NW
CHUNK = 128
NCH = BPW // CHUNK
GROUPS = BPW // L


def _mf_body(u_hbm, i_hbm, up_hbm, iq_hbm, ub_hbm, ib_hbm, mu_hbm, out_hbm,
             uidx, iidx, up_rows, iq_rows, ubv, ibv, outv, muv, sem):
    c = lax.axis_index("c")
    s = lax.axis_index("s")
    wid = s * NC + c

    pltpu.sync_copy(u_hbm.at[pl.ds(wid * NCH, NCH)], uidx)
    pltpu.sync_copy(i_hbm.at[pl.ds(wid * NCH, NCH)], iidx)
    pltpu.sync_copy(mu_hbm, muv)

    copies = []
    for ci in range(NCH):
        sl = pl.ds(ci * CHUNK, CHUNK)
        copies.append(pltpu.async_copy(up_hbm.at[uidx.at[ci]], up_rows.at[sl], sem))
        copies.append(pltpu.async_copy(iq_hbm.at[iidx.at[ci]], iq_rows.at[sl], sem))
        copies.append(pltpu.async_copy(ub_hbm.at[uidx.at[ci]], ubv.at[sl], sem))
        copies.append(pltpu.async_copy(ib_hbm.at[iidx.at[ci]], ibv.at[sl], sem))
    for cp in copies:
        cp.wait()

    mu_s = muv[...]
    lane = lax.broadcasted_iota(jnp.int32, (L,), 0)

    def group_body(g, carry):
        rows = lane + g * L

        def d_body(d, acc):
            col = jnp.full((L,), d, jnp.int32)
            upv = plsc.load_gather(up_rows, [rows, col])
            iqv = plsc.load_gather(iq_rows, [rows, col])
            return acc + upv * iqv

        acc = lax.fori_loop(0, D, d_body, jnp.zeros((L,), jnp.float32),
                            unroll=8)
        ubg = ubv[pl.ds(g * L, L)]
        ibg = ibv[pl.ds(g * L, L)]
        outv[pl.ds(g * L, L)] = acc + ubg + ibg + mu_s
        return carry

    lax.fori_loop(0, GROUPS, group_body, 0)
    pltpu.sync_copy(outv, out_hbm.at[pl.ds(wid * BPW, BPW)])


@jax.jit
def kernel(u, i, user_p, item_q, user_b, item_b, mu):
    u2 = u.reshape(B // CHUNK, CHUNK)
    i2 = i.reshape(B // CHUNK, CHUNK)
    mu16 = jnp.broadcast_to(mu, (L,))
    ub1 = user_b.reshape(-1)
    ib1 = item_b.reshape(-1)
    mesh = plsc.VectorSubcoreMesh(core_axis_name="c", subcore_axis_name="s",
                                  num_cores=NC, num_subcores=NS)
    fn = pl.kernel(
        _mf_body,
        out_type=jax.ShapeDtypeStruct((B,), jnp.float32),
        mesh=mesh,
        compiler_params=pltpu.CompilerParams(needs_layout_passes=False,
                                             use_tc_tiling_on_sc=False),
        scratch_types=[
            pltpu.VMEM((NCH, CHUNK), jnp.int32),
            pltpu.VMEM((NCH, CHUNK), jnp.int32),
            pltpu.VMEM((BPW, D), jnp.float32),
            pltpu.VMEM((BPW, D), jnp.float32),
            pltpu.VMEM((BPW,), jnp.float32),
            pltpu.VMEM((BPW,), jnp.float32),
            pltpu.VMEM((BPW,), jnp.float32),
            pltpu.VMEM((L,), jnp.float32),
            pltpu.SemaphoreType.DMA,
        ],
    )
    return fn(u2, i2, user_p, item_q, ub1, ib1, mu16)
